# Initial kernel scaffold; baseline (speedup 1.0000x reference)
import jax
import jax.numpy as jnp
from jax.experimental import pallas as pl


def kernel(h, E, tokens, tied_scale_act, tied_scale_time, proto_scale_act, proto_scale_time, proto_prior_act, proto_prior_time, proto_temp_act, proto_temp_time):
    raise NotImplementedError("write your pallas kernel here")


# speedup vs baseline: 97.0787x; 97.0787x over previous
import functools

import jax
import jax.numpy as jnp
from jax import lax
from jax.experimental import pallas as pl
from jax.experimental.pallas import tpu as pltpu

_SPECIAL = 4
_LABEL_ID = 1
_ACT_V = 64
_TIME_V = 32
_C = _ACT_V + _TIME_V
_BLK = 256

_HI = jax.lax.Precision.HIGHEST


def _proto_kernel(tok_ref, nxt_ref, h_ref, e_ref, params_ref,
                  out_ref, v_ref, sq_ref):
    q = pl.program_id(1)
    blk = h_ref.shape[1]

    s_ta = params_ref[0]
    s_tt = params_ref[1]
    s_pa = params_ref[2]
    s_pt = params_ref[3]
    alpha_a = params_ref[4]
    alpha_t = params_ref[5]
    tau_a = params_ref[6]
    tau_t = params_ref[7]

    col96 = lax.broadcasted_iota(jnp.int32, (1, _C), 1)
    is_act_col = col96 < _ACT_V

    @pl.when(q == 0)
    def _init():
        e = e_ref[...]
        en = e / jnp.maximum(
            jnp.sqrt(jnp.sum(e * e, axis=1, keepdims=True)), 1e-12)
        row_idx = lax.broadcasted_iota(jnp.int32, (_C, 1), 0)
        alpha_row = jnp.where(row_idx < _ACT_V, alpha_a, alpha_t)
        v_ref[...] = alpha_row * en
        col128 = lax.broadcasted_iota(jnp.int32, (1, 128), 1)
        alpha_col = jnp.where(col128 < _ACT_V, alpha_a, alpha_t)
        sq_ref[...] = jnp.where(col128 < _C, alpha_col * alpha_col, 0.0)

    h = h_ref[0]
    hss = jnp.sum(h * h, axis=1, keepdims=True)
    hn = h / jnp.maximum(jnp.sqrt(hss), 1e-12)
    hn_ss = jnp.sum(hn * hn, axis=1, keepdims=True)

    tok = tok_ref[0]
    nxt = nxt_ref[0]
    is_label = tok == _LABEL_ID
    sup_a = is_label & (nxt >= _SPECIAL) & (nxt < _SPECIAL + _ACT_V)
    sup_t = is_label & (nxt >= _SPECIAL + _ACT_V) & (nxt < _SPECIAL + _C)
    sup = sup_a | sup_t
    colc = lax.broadcasted_iota(jnp.int32, (blk, _C), 1)
    m = jnp.where(sup & (colc == nxt - _SPECIAL), 1.0, 0.0)

    num = lax.dot_general(hn, v_ref[...], (((1,), (1,)), ((), ())),
                          precision=_HI)
    gram = lax.dot_general(hn, hn, (((1,), (1,)), ((), ())),
                           precision=_HI)
    rowi = lax.broadcasted_iota(jnp.int32, (blk, blk), 0)
    coli = lax.broadcasted_iota(jnp.int32, (blk, blk), 1)
    strict = coli < rowi
    gram_l = jnp.where(strict, gram, 0.0)
    num = num + lax.dot_general(gram_l, m, (((1,), (0,)), ((), ())),
                                precision=_HI)

    inc = m * (2.0 * num + hn_ss)
    sup_af = jnp.where(sup_a, 1.0, 0.0)
    sup_tf = jnp.where(sup_t, 1.0, 0.0)
    x = jnp.concatenate(
        [inc, sup_af, sup_tf, jnp.zeros((blk, 128 - _C - 2), jnp.float32)],
        axis=1)
    ones_l = jnp.where(strict, 1.0, 0.0)
    cum = lax.dot_general(ones_l, x, (((1,), (0,)), ((), ())),
                          precision=_HI)
    base = sq_ref[...]
    sqnorm = base[:, :_C] + cum[:, :_C]
    cnts = base[:, _C:_C + 2] + cum[:, _C:_C + 2]

    denom = jnp.maximum(jnp.sqrt(jnp.maximum(sqnorm, 0.0)), 1e-12)
    gate_cnt = jnp.where(is_act_col, cnts[:, 0:1], cnts[:, 1:2])
    gate = is_label & (gate_cnt > 0.0)
    tau_col = jnp.where(is_act_col, tau_a, tau_t)
    proto = jnp.where(gate, num / denom * tau_col, 0.0)

    tied = lax.dot_general(h, e_ref[...], (((1,), (1,)), ((), ())),
                           precision=_HI)
    s_tied_col = jnp.where(is_act_col, s_ta, s_tt)
    s_proto_col = jnp.where(is_act_col, s_pa, s_pt)
    out_ref[0] = s_tied_col * tied + s_proto_col * proto

    v_ref[...] = v_ref[...] + lax.dot_general(
        m, hn, (((0,), (0,)), ((), ())), precision=_HI)
    sq_ref[...] = base + jnp.sum(x, axis=0, keepdims=True)


def kernel(h, E, tokens, tied_scale_act, tied_scale_time, proto_scale_act,
           proto_scale_time, proto_prior_act, proto_prior_time,
           proto_temp_act, proto_temp_time):
    b, t, d = h.shape
    blk = _BLK

    params = jnp.stack([
        jax.nn.softplus(tied_scale_act),
        jax.nn.softplus(tied_scale_time),
        jax.nn.softplus(proto_scale_act),
        jax.nn.softplus(proto_scale_time),
        jax.nn.softplus(proto_prior_act),
        jax.nn.softplus(proto_prior_time),
        jax.nn.softplus(proto_temp_act),
        jax.nn.softplus(proto_temp_time),
    ]).astype(jnp.float32)

    tokens = tokens.astype(jnp.int32)
    nxt = jnp.roll(tokens, -1, axis=1)
    tok3 = tokens.reshape(b, t, 1)
    nxt3 = nxt.reshape(b, t, 1)
    e_sub = E[_SPECIAL:_SPECIAL + _C].astype(jnp.float32)

    grid = (b, t // blk)
    out = pl.pallas_call(
        _proto_kernel,
        grid=grid,
        in_specs=[
            pl.BlockSpec((1, blk, 1), lambda i, j: (i, j, 0)),
            pl.BlockSpec((1, blk, 1), lambda i, j: (i, j, 0)),
            pl.BlockSpec((1, blk, d), lambda i, j: (i, j, 0)),
            pl.BlockSpec((_C, d), lambda i, j: (0, 0)),
            pl.BlockSpec(memory_space=pltpu.SMEM),
        ],
        out_specs=pl.BlockSpec((1, blk, _C), lambda i, j: (i, j, 0)),
        out_shape=jax.ShapeDtypeStruct((b, t, _C), jnp.float32),
        scratch_shapes=[
            pltpu.VMEM((_C, d), jnp.float32),
            pltpu.VMEM((1, 128), jnp.float32),
        ],
    )(tok3, nxt3, h.astype(jnp.float32), e_sub, params)
    return out


# speedup vs baseline: 242.2265x; 2.4952x over previous
import functools

import jax
import jax.numpy as jnp
from jax import lax
from jax.experimental import pallas as pl
from jax.experimental.pallas import tpu as pltpu

_SPECIAL = 4
_LABEL_ID = 1
_ACT_V = 64
_TIME_V = 32
_C = _ACT_V + _TIME_V
_BLK = 256

_HI = jax.lax.Precision.DEFAULT


def _proto_kernel(tok_ref, nxt_ref, h_ref, e_ref, params_ref,
                  out_ref, v_ref, sq_ref, l_ref):
    b = pl.program_id(0)
    q = pl.program_id(1)
    blk = h_ref.shape[1]

    s_ta = params_ref[0]
    s_tt = params_ref[1]
    s_pa = params_ref[2]
    s_pt = params_ref[3]
    alpha_a = params_ref[4]
    alpha_t = params_ref[5]
    tau_a = params_ref[6]
    tau_t = params_ref[7]

    col96 = lax.broadcasted_iota(jnp.int32, (1, _C), 1)
    is_act_col = col96 < _ACT_V

    @pl.when(q == 0)
    def _init():
        e = e_ref[...]
        en = e / jnp.maximum(
            jnp.sqrt(jnp.sum(e * e, axis=1, keepdims=True)), 1e-12)
        row_idx = lax.broadcasted_iota(jnp.int32, (_C, 1), 0)
        alpha_row = jnp.where(row_idx < _ACT_V, alpha_a, alpha_t)
        v_ref[...] = alpha_row * en
        col128 = lax.broadcasted_iota(jnp.int32, (1, 128), 1)
        alpha_col = jnp.where(col128 < _ACT_V, alpha_a, alpha_t)
        sq_ref[...] = jnp.where(col128 < _C, alpha_col * alpha_col, 0.0)

    @pl.when((b == 0) & (q == 0))
    def _init_mask():
        rowi = lax.broadcasted_iota(jnp.int32, (blk, blk), 0)
        coli = lax.broadcasted_iota(jnp.int32, (blk, blk), 1)
        l_ref[...] = jnp.where(coli < rowi, 1.0, 0.0)

    h = h_ref[0]
    hss = jnp.sum(h * h, axis=1, keepdims=True)
    hn = h / jnp.maximum(jnp.sqrt(hss), 1e-12)
    hn_ss = jnp.sum(hn * hn, axis=1, keepdims=True)

    tok = tok_ref[0]
    nxt = nxt_ref[0]
    is_label = tok == _LABEL_ID
    sup_a = is_label & (nxt >= _SPECIAL) & (nxt < _SPECIAL + _ACT_V)
    sup_t = is_label & (nxt >= _SPECIAL + _ACT_V) & (nxt < _SPECIAL + _C)
    sup = sup_a | sup_t
    colc = lax.broadcasted_iota(jnp.int32, (blk, _C), 1)
    m = jnp.where(sup & (colc == nxt - _SPECIAL), 1.0, 0.0)

    num = lax.dot_general(hn, v_ref[...], (((1,), (1,)), ((), ())),
                          precision=_HI)
    gram = lax.dot_general(hn, hn, (((1,), (1,)), ((), ())),
                           precision=_HI)
    ones_l = l_ref[...]
    gram_l = gram * ones_l
    num = num + lax.dot_general(gram_l, m, (((1,), (0,)), ((), ())),
                                precision=_HI)

    inc = m * (2.0 * num + hn_ss)
    sup_af = jnp.where(sup_a, 1.0, 0.0)
    sup_tf = jnp.where(sup_t, 1.0, 0.0)
    x = jnp.concatenate(
        [inc, sup_af, sup_tf, jnp.zeros((blk, 128 - _C - 2), jnp.float32)],
        axis=1)
    cum = lax.dot_general(ones_l, x, (((1,), (0,)), ((), ())),
                          precision=_HI)
    base = sq_ref[...]
    sqnorm = base[:, :_C] + cum[:, :_C]
    cnts = base[:, _C:_C + 2] + cum[:, _C:_C + 2]

    denom = jnp.maximum(jnp.sqrt(jnp.maximum(sqnorm, 0.0)), 1e-12)
    gate_cnt = jnp.where(is_act_col, cnts[:, 0:1], cnts[:, 1:2])
    gate = is_label & (gate_cnt > 0.0)
    tau_col = jnp.where(is_act_col, tau_a, tau_t)
    proto = jnp.where(gate, num / denom * tau_col, 0.0)

    tied = lax.dot_general(h, e_ref[...], (((1,), (1,)), ((), ())),
                           precision=_HI)
    s_tied_col = jnp.where(is_act_col, s_ta, s_tt)
    s_proto_col = jnp.where(is_act_col, s_pa, s_pt)
    out_ref[0] = s_tied_col * tied + s_proto_col * proto

    v_ref[...] = v_ref[...] + lax.dot_general(
        m, hn, (((0,), (0,)), ((), ())), precision=_HI)
    sq_ref[...] = base + jnp.sum(x, axis=0, keepdims=True)


def kernel(h, E, tokens, tied_scale_act, tied_scale_time, proto_scale_act,
           proto_scale_time, proto_prior_act, proto_prior_time,
           proto_temp_act, proto_temp_time):
    b, t, d = h.shape
    blk = _BLK

    params = jnp.stack([
        jax.nn.softplus(tied_scale_act),
        jax.nn.softplus(tied_scale_time),
        jax.nn.softplus(proto_scale_act),
        jax.nn.softplus(proto_scale_time),
        jax.nn.softplus(proto_prior_act),
        jax.nn.softplus(proto_prior_time),
        jax.nn.softplus(proto_temp_act),
        jax.nn.softplus(proto_temp_time),
    ]).astype(jnp.float32)

    tokens = tokens.astype(jnp.int32)
    nxt = jnp.roll(tokens, -1, axis=1)
    tok3 = tokens.reshape(b, t, 1)
    nxt3 = nxt.reshape(b, t, 1)
    e_sub = E[_SPECIAL:_SPECIAL + _C].astype(jnp.float32)

    grid = (b, t // blk)
    out = pl.pallas_call(
        _proto_kernel,
        grid=grid,
        in_specs=[
            pl.BlockSpec((1, blk, 1), lambda i, j: (i, j, 0)),
            pl.BlockSpec((1, blk, 1), lambda i, j: (i, j, 0)),
            pl.BlockSpec((1, blk, d), lambda i, j: (i, j, 0)),
            pl.BlockSpec((_C, d), lambda i, j: (0, 0)),
            pl.BlockSpec(memory_space=pltpu.SMEM),
        ],
        out_specs=pl.BlockSpec((1, blk, _C), lambda i, j: (i, j, 0)),
        out_shape=jax.ShapeDtypeStruct((b, t, _C), jnp.float32),
        scratch_shapes=[
            pltpu.VMEM((_C, d), jnp.float32),
            pltpu.VMEM((1, 128), jnp.float32),
            pltpu.VMEM((blk, blk), jnp.float32),
        ],
    )(tok3, nxt3, h.astype(jnp.float32), e_sub, params)
    return out


# speedup vs baseline: 254.7649x; 1.0518x over previous
import jax
import jax.numpy as jnp
from jax import lax
from jax.experimental import pallas as pl
from jax.experimental.pallas import tpu as pltpu

_SPECIAL = 4
_LABEL_ID = 1
_ACT_V = 64
_TIME_V = 32
_C = _ACT_V + _TIME_V
_BLK = 256

_PREC = jax.lax.Precision.DEFAULT


def _proto_kernel(tok_ref, nxt_ref, h_ref, e_ref, params_ref,
                  out_ref, w_ref, sq_ref, l_ref):
    b = pl.program_id(0)
    q = pl.program_id(1)
    blk = h_ref.shape[1]

    s_ta = params_ref[0]
    s_tt = params_ref[1]
    s_pa = params_ref[2]
    s_pt = params_ref[3]
    alpha_a = params_ref[4]
    alpha_t = params_ref[5]
    tau_a = params_ref[6]
    tau_t = params_ref[7]

    col96 = lax.broadcasted_iota(jnp.int32, (1, _C), 1)
    is_act_col = col96 < _ACT_V

    @pl.when((b == 0) & (q == 0))
    def _init_once():
        rowi = lax.broadcasted_iota(jnp.int32, (blk, blk), 0)
        coli = lax.broadcasted_iota(jnp.int32, (blk, blk), 1)
        l_ref[...] = jnp.where(coli < rowi, 1.0, 0.0)
        w_ref[_C:2 * _C, :] = e_ref[...]

    @pl.when(q == 0)
    def _init_seq():
        e = e_ref[...]
        en = e / jnp.maximum(
            jnp.sqrt(jnp.sum(e * e, axis=1, keepdims=True)), 1e-12)
        row_idx = lax.broadcasted_iota(jnp.int32, (_C, 1), 0)
        alpha_row = jnp.where(row_idx < _ACT_V, alpha_a, alpha_t)
        w_ref[0:_C, :] = alpha_row * en
        col128 = lax.broadcasted_iota(jnp.int32, (1, 128), 1)
        alpha_col = jnp.where(col128 < _ACT_V, alpha_a, alpha_t)
        sq_ref[...] = jnp.where(col128 < _C, alpha_col * alpha_col, 0.0)

    h = h_ref[0]
    hss = jnp.sum(h * h, axis=1, keepdims=True)
    inv = 1.0 / jnp.maximum(jnp.sqrt(hss), 1e-12)
    hn_ss = hss * inv * inv

    tok = tok_ref[0]
    nxt = nxt_ref[0]
    is_label = tok == _LABEL_ID
    sup_a = is_label & (nxt >= _SPECIAL) & (nxt < _SPECIAL + _ACT_V)
    sup_t = is_label & (nxt >= _SPECIAL + _ACT_V) & (nxt < _SPECIAL + _C)
    sup = sup_a | sup_t
    colc = lax.broadcasted_iota(jnp.int32, (blk, _C), 1)
    m = jnp.where(sup & (colc == nxt - _SPECIAL), 1.0, 0.0)
    mi = m * inv

    y = lax.dot_general(h, w_ref[...], (((1,), (1,)), ((), ())),
                        precision=_PREC)
    tied = y[:, _C:2 * _C]
    gram = lax.dot_general(h, h, (((1,), (1,)), ((), ())),
                           precision=_PREC)
    gram_l = gram * l_ref[...]
    num = inv * (y[:, :_C] +
                 lax.dot_general(gram_l, mi, (((1,), (0,)), ((), ())),
                                 precision=_PREC))

    inc = m * (2.0 * num + hn_ss)
    sup_af = jnp.where(sup_a, 1.0, 0.0)
    sup_tf = jnp.where(sup_t, 1.0, 0.0)
    x = jnp.concatenate(
        [inc, sup_af, sup_tf, jnp.zeros((blk, 128 - _C - 2), jnp.float32)],
        axis=1)
    cum = lax.dot_general(l_ref[...], x, (((1,), (0,)), ((), ())),
                          precision=_PREC)
    base = sq_ref[...]
    sqnorm = base[:, :_C] + cum[:, :_C]
    cnts = base[:, _C:_C + 2] + cum[:, _C:_C + 2]

    denom = jnp.maximum(jnp.sqrt(jnp.maximum(sqnorm, 0.0)), 1e-12)
    gate_cnt = jnp.where(is_act_col, cnts[:, 0:1], cnts[:, 1:2])
    gate = is_label & (gate_cnt > 0.0)
    tau_col = jnp.where(is_act_col, tau_a, tau_t)
    proto = jnp.where(gate, num / denom * tau_col, 0.0)

    s_tied_col = jnp.where(is_act_col, s_ta, s_tt)
    s_proto_col = jnp.where(is_act_col, s_pa, s_pt)
    out_ref[0] = s_tied_col * tied + s_proto_col * proto

    w_ref[0:_C, :] = w_ref[0:_C, :] + lax.dot_general(
        mi, h, (((0,), (0,)), ((), ())), precision=_PREC)
    sq_ref[...] = base + jnp.sum(x, axis=0, keepdims=True)


def kernel(h, E, tokens, tied_scale_act, tied_scale_time, proto_scale_act,
           proto_scale_time, proto_prior_act, proto_prior_time,
           proto_temp_act, proto_temp_time):
    b, t, d = h.shape
    blk = _BLK

    params = jnp.stack([
        jax.nn.softplus(tied_scale_act),
        jax.nn.softplus(tied_scale_time),
        jax.nn.softplus(proto_scale_act),
        jax.nn.softplus(proto_scale_time),
        jax.nn.softplus(proto_prior_act),
        jax.nn.softplus(proto_prior_time),
        jax.nn.softplus(proto_temp_act),
        jax.nn.softplus(proto_temp_time),
    ]).astype(jnp.float32)

    tokens = tokens.astype(jnp.int32)
    nxt = jnp.roll(tokens, -1, axis=1)
    tok3 = tokens.reshape(b, t, 1)
    nxt3 = nxt.reshape(b, t, 1)
    e_sub = E[_SPECIAL:_SPECIAL + _C].astype(jnp.float32)

    grid = (b, t // blk)
    out = pl.pallas_call(
        _proto_kernel,
        grid=grid,
        in_specs=[
            pl.BlockSpec((1, blk, 1), lambda i, j: (i, j, 0)),
            pl.BlockSpec((1, blk, 1), lambda i, j: (i, j, 0)),
            pl.BlockSpec((1, blk, d), lambda i, j: (i, j, 0)),
            pl.BlockSpec((_C, d), lambda i, j: (0, 0)),
            pl.BlockSpec(memory_space=pltpu.SMEM),
        ],
        out_specs=pl.BlockSpec((1, blk, _C), lambda i, j: (i, j, 0)),
        out_shape=jax.ShapeDtypeStruct((b, t, _C), jnp.float32),
        scratch_shapes=[
            pltpu.VMEM((2 * _C, d), jnp.float32),
            pltpu.VMEM((1, 128), jnp.float32),
            pltpu.VMEM((blk, blk), jnp.float32),
        ],
    )(tok3, nxt3, h.astype(jnp.float32), e_sub, params)
    return out


# speedup vs baseline: 286.9272x; 1.1262x over previous
import jax
import jax.numpy as jnp
from jax import lax
from jax.experimental import pallas as pl
from jax.experimental.pallas import tpu as pltpu

_SPECIAL = 4
_LABEL_ID = 1
_ACT_V = 64
_TIME_V = 32
_C = _ACT_V + _TIME_V
_BLK = 512

_PREC = jax.lax.Precision.DEFAULT


def _proto_kernel(tok_ref, nxt_ref, h_ref, e_ref, params_ref,
                  out_ref, w_ref, sq_ref, l_ref):
    b = pl.program_id(0)
    q = pl.program_id(1)
    blk = h_ref.shape[1]

    s_ta = params_ref[0]
    s_tt = params_ref[1]
    s_pa = params_ref[2]
    s_pt = params_ref[3]
    alpha_a = params_ref[4]
    alpha_t = params_ref[5]
    tau_a = params_ref[6]
    tau_t = params_ref[7]

    col96 = lax.broadcasted_iota(jnp.int32, (1, _C), 1)
    is_act_col = col96 < _ACT_V

    @pl.when((b == 0) & (q == 0))
    def _init_once():
        rowi = lax.broadcasted_iota(jnp.int32, (blk, blk), 0)
        coli = lax.broadcasted_iota(jnp.int32, (blk, blk), 1)
        l_ref[...] = jnp.where(coli < rowi, 1.0, 0.0)
        w_ref[_C:2 * _C, :] = e_ref[...]

    @pl.when(q == 0)
    def _init_seq():
        e = e_ref[...]
        en = e / jnp.maximum(
            jnp.sqrt(jnp.sum(e * e, axis=1, keepdims=True)), 1e-12)
        row_idx = lax.broadcasted_iota(jnp.int32, (_C, 1), 0)
        alpha_row = jnp.where(row_idx < _ACT_V, alpha_a, alpha_t)
        w_ref[0:_C, :] = alpha_row * en
        col128 = lax.broadcasted_iota(jnp.int32, (1, 128), 1)
        alpha_col = jnp.where(col128 < _ACT_V, alpha_a, alpha_t)
        sq_ref[...] = jnp.where(col128 < _C, alpha_col * alpha_col, 0.0)

    h = h_ref[0]
    hss = jnp.sum(h * h, axis=1, keepdims=True)
    inv = 1.0 / jnp.maximum(jnp.sqrt(hss), 1e-12)
    hn_ss = hss * inv * inv

    tok = tok_ref[0]
    nxt = nxt_ref[0]
    is_label = tok == _LABEL_ID
    sup_a = is_label & (nxt >= _SPECIAL) & (nxt < _SPECIAL + _ACT_V)
    sup_t = is_label & (nxt >= _SPECIAL + _ACT_V) & (nxt < _SPECIAL + _C)
    sup = sup_a | sup_t
    colc = lax.broadcasted_iota(jnp.int32, (blk, _C), 1)
    m = jnp.where(sup & (colc == nxt - _SPECIAL), 1.0, 0.0)
    mi = m * inv

    y = lax.dot_general(h, w_ref[...], (((1,), (1,)), ((), ())),
                        precision=_PREC)
    tied = y[:, _C:2 * _C]
    gram = lax.dot_general(h, h, (((1,), (1,)), ((), ())),
                           precision=_PREC)
    gram_l = gram * l_ref[...]
    num = inv * (y[:, :_C] +
                 lax.dot_general(gram_l, mi, (((1,), (0,)), ((), ())),
                                 precision=_PREC))

    inc = m * (2.0 * num + hn_ss)
    sup_af = jnp.where(sup_a, 1.0, 0.0)
    sup_tf = jnp.where(sup_t, 1.0, 0.0)
    x = jnp.concatenate(
        [inc, sup_af, sup_tf, jnp.zeros((blk, 128 - _C - 2), jnp.float32)],
        axis=1)
    cum = lax.dot_general(l_ref[...], x, (((1,), (0,)), ((), ())),
                          precision=_PREC)
    base = sq_ref[...]
    sqnorm = base[:, :_C] + cum[:, :_C]
    cnts = base[:, _C:_C + 2] + cum[:, _C:_C + 2]

    denom = jnp.maximum(jnp.sqrt(jnp.maximum(sqnorm, 0.0)), 1e-12)
    gate_cnt = jnp.where(is_act_col, cnts[:, 0:1], cnts[:, 1:2])
    gate = is_label & (gate_cnt > 0.0)
    tau_col = jnp.where(is_act_col, tau_a, tau_t)
    proto = jnp.where(gate, num / denom * tau_col, 0.0)

    s_tied_col = jnp.where(is_act_col, s_ta, s_tt)
    s_proto_col = jnp.where(is_act_col, s_pa, s_pt)
    out_ref[0] = s_tied_col * tied + s_proto_col * proto

    w_ref[0:_C, :] = w_ref[0:_C, :] + lax.dot_general(
        mi, h, (((0,), (0,)), ((), ())), precision=_PREC)
    sq_ref[...] = base + jnp.sum(x, axis=0, keepdims=True)


def kernel(h, E, tokens, tied_scale_act, tied_scale_time, proto_scale_act,
           proto_scale_time, proto_prior_act, proto_prior_time,
           proto_temp_act, proto_temp_time):
    b, t, d = h.shape
    blk = _BLK

    params = jnp.stack([
        jax.nn.softplus(tied_scale_act),
        jax.nn.softplus(tied_scale_time),
        jax.nn.softplus(proto_scale_act),
        jax.nn.softplus(proto_scale_time),
        jax.nn.softplus(proto_prior_act),
        jax.nn.softplus(proto_prior_time),
        jax.nn.softplus(proto_temp_act),
        jax.nn.softplus(proto_temp_time),
    ]).astype(jnp.float32)

    tokens = tokens.astype(jnp.int32)
    nxt = jnp.roll(tokens, -1, axis=1)
    tok3 = tokens.reshape(b, t, 1)
    nxt3 = nxt.reshape(b, t, 1)
    e_sub = E[_SPECIAL:_SPECIAL + _C].astype(jnp.float32)

    grid = (b, t // blk)
    out = pl.pallas_call(
        _proto_kernel,
        grid=grid,
        in_specs=[
            pl.BlockSpec((1, blk, 1), lambda i, j: (i, j, 0)),
            pl.BlockSpec((1, blk, 1), lambda i, j: (i, j, 0)),
            pl.BlockSpec((1, blk, d), lambda i, j: (i, j, 0)),
            pl.BlockSpec((_C, d), lambda i, j: (0, 0)),
            pl.BlockSpec(memory_space=pltpu.SMEM),
        ],
        out_specs=pl.BlockSpec((1, blk, _C), lambda i, j: (i, j, 0)),
        out_shape=jax.ShapeDtypeStruct((b, t, _C), jnp.float32),
        scratch_shapes=[
            pltpu.VMEM((2 * _C, d), jnp.float32),
            pltpu.VMEM((1, 128), jnp.float32),
            pltpu.VMEM((blk, blk), jnp.float32),
        ],
    )(tok3, nxt3, h.astype(jnp.float32), e_sub, params)
    return out


# speedup vs baseline: 296.6687x; 1.0340x over previous
import jax
import jax.numpy as jnp
from jax import lax
from jax.experimental import pallas as pl
from jax.experimental.pallas import tpu as pltpu

_SPECIAL = 4
_LABEL_ID = 1
_ACT_V = 64
_TIME_V = 32
_C = _ACT_V + _TIME_V
_BLK = 512

_PREC = jax.lax.Precision.DEFAULT


def _proto_kernel(tok_ref, nxt_ref, h_ref, e_ref, params_ref,
                  out_ref, w_ref, sq_ref, l_ref):
    b = pl.program_id(0)
    blk = _BLK
    t_len = h_ref.shape[1]

    s_ta = params_ref[0]
    s_tt = params_ref[1]
    s_pa = params_ref[2]
    s_pt = params_ref[3]
    alpha_a = params_ref[4]
    alpha_t = params_ref[5]
    tau_a = params_ref[6]
    tau_t = params_ref[7]

    col96 = lax.broadcasted_iota(jnp.int32, (1, _C), 1)
    is_act_col = col96 < _ACT_V
    tau_col = jnp.where(is_act_col, tau_a, tau_t)
    s_tied_col = jnp.where(is_act_col, s_ta, s_tt)
    s_proto_col = jnp.where(is_act_col, s_pa, s_pt)

    @pl.when(b == 0)
    def _init_once():
        rowi = lax.broadcasted_iota(jnp.int32, (blk, blk), 0)
        coli = lax.broadcasted_iota(jnp.int32, (blk, blk), 1)
        l_ref[...] = jnp.where(coli < rowi, 1.0, 0.0)
        w_ref[_C:2 * _C, :] = e_ref[...]

    e = e_ref[...]
    en = e / jnp.maximum(
        jnp.sqrt(jnp.sum(e * e, axis=1, keepdims=True)), 1e-12)
    row_idx = lax.broadcasted_iota(jnp.int32, (_C, 1), 0)
    alpha_row = jnp.where(row_idx < _ACT_V, alpha_a, alpha_t)
    w_ref[0:_C, :] = alpha_row * en
    col128 = lax.broadcasted_iota(jnp.int32, (1, 128), 1)
    alpha_col = jnp.where(col128 < _ACT_V, alpha_a, alpha_t)
    sq_ref[...] = jnp.where(col128 < _C, alpha_col * alpha_col, 0.0)

    ones_l = l_ref[...]

    for j in range(t_len // blk):
        sl = slice(j * blk, (j + 1) * blk)
        h = h_ref[0, sl, :]
        hss = jnp.sum(h * h, axis=1, keepdims=True)
        inv = 1.0 / jnp.maximum(jnp.sqrt(hss), 1e-12)
        hn_ss = hss * inv * inv

        tok = tok_ref[0, sl, :]
        nxt = nxt_ref[0, sl, :]
        is_label = tok == _LABEL_ID
        sup_a = is_label & (nxt >= _SPECIAL) & (nxt < _SPECIAL + _ACT_V)
        sup_t = is_label & (nxt >= _SPECIAL + _ACT_V) & (nxt < _SPECIAL + _C)
        sup = sup_a | sup_t
        colc = lax.broadcasted_iota(jnp.int32, (blk, _C), 1)
        m = jnp.where(sup & (colc == nxt - _SPECIAL), 1.0, 0.0)
        mi = m * inv

        y = lax.dot_general(h, w_ref[...], (((1,), (1,)), ((), ())),
                            precision=_PREC)
        tied = y[:, _C:2 * _C]
        gram = lax.dot_general(h, h, (((1,), (1,)), ((), ())),
                               precision=_PREC)
        gram_l = gram * ones_l
        num = inv * (y[:, :_C] +
                     lax.dot_general(gram_l, mi, (((1,), (0,)), ((), ())),
                                     precision=_PREC))

        inc = m * (2.0 * num + hn_ss)
        sup_af = jnp.where(sup_a, 1.0, 0.0)
        sup_tf = jnp.where(sup_t, 1.0, 0.0)
        x = jnp.concatenate(
            [inc, sup_af, sup_tf,
             jnp.zeros((blk, 128 - _C - 2), jnp.float32)],
            axis=1)
        cum = lax.dot_general(ones_l, x, (((1,), (0,)), ((), ())),
                              precision=_PREC)
        base = sq_ref[...]
        sqnorm = base[:, :_C] + cum[:, :_C]
        cnts = base[:, _C:_C + 2] + cum[:, _C:_C + 2]

        denom = jnp.maximum(jnp.sqrt(jnp.maximum(sqnorm, 0.0)), 1e-12)
        gate_cnt = jnp.where(is_act_col, cnts[:, 0:1], cnts[:, 1:2])
        gate = is_label & (gate_cnt > 0.0)
        proto = jnp.where(gate, num / denom * tau_col, 0.0)

        out_ref[0, sl, :] = s_tied_col * tied + s_proto_col * proto

        w_ref[0:_C, :] = w_ref[0:_C, :] + lax.dot_general(
            mi, h, (((0,), (0,)), ((), ())), precision=_PREC)
        sq_ref[...] = base + jnp.sum(x, axis=0, keepdims=True)


def kernel(h, E, tokens, tied_scale_act, tied_scale_time, proto_scale_act,
           proto_scale_time, proto_prior_act, proto_prior_time,
           proto_temp_act, proto_temp_time):
    b, t, d = h.shape

    params = jnp.stack([
        jax.nn.softplus(tied_scale_act),
        jax.nn.softplus(tied_scale_time),
        jax.nn.softplus(proto_scale_act),
        jax.nn.softplus(proto_scale_time),
        jax.nn.softplus(proto_prior_act),
        jax.nn.softplus(proto_prior_time),
        jax.nn.softplus(proto_temp_act),
        jax.nn.softplus(proto_temp_time),
    ]).astype(jnp.float32)

    tokens = tokens.astype(jnp.int32)
    nxt = jnp.roll(tokens, -1, axis=1)
    tok3 = tokens.reshape(b, t, 1)
    nxt3 = nxt.reshape(b, t, 1)
    e_sub = E[_SPECIAL:_SPECIAL + _C].astype(jnp.float32)

    out = pl.pallas_call(
        _proto_kernel,
        grid=(b,),
        in_specs=[
            pl.BlockSpec((1, t, 1), lambda i: (i, 0, 0)),
            pl.BlockSpec((1, t, 1), lambda i: (i, 0, 0)),
            pl.BlockSpec((1, t, d), lambda i: (i, 0, 0)),
            pl.BlockSpec((_C, d), lambda i: (0, 0)),
            pl.BlockSpec(memory_space=pltpu.SMEM),
        ],
        out_specs=pl.BlockSpec((1, t, _C), lambda i: (i, 0, 0)),
        out_shape=jax.ShapeDtypeStruct((b, t, _C), jnp.float32),
        scratch_shapes=[
            pltpu.VMEM((2 * _C, d), jnp.float32),
            pltpu.VMEM((1, 128), jnp.float32),
            pltpu.VMEM((_BLK, _BLK), jnp.float32),
        ],
    )(tok3, nxt3, h.astype(jnp.float32), e_sub, params)
    return out


# speedup vs baseline: 297.2052x; 1.0018x over previous
import jax
import jax.numpy as jnp
from jax import lax
from jax.experimental import pallas as pl
from jax.experimental.pallas import tpu as pltpu

_SPECIAL = 4
_LABEL_ID = 1
_ACT_V = 64
_TIME_V = 32
_C = _ACT_V + _TIME_V
_BLK = 512

_PREC = jax.lax.Precision.DEFAULT


def _proto_kernel(tok_ref, nxt_ref, h_ref, e_ref, params_ref,
                  out_ref, w_ref, sq_ref, l_ref, lb_ref):
    b = pl.program_id(0)
    blk = _BLK
    t_len = h_ref.shape[1]

    s_ta = params_ref[0]
    s_tt = params_ref[1]
    s_pa = params_ref[2]
    s_pt = params_ref[3]
    alpha_a = params_ref[4]
    alpha_t = params_ref[5]
    tau_a = params_ref[6]
    tau_t = params_ref[7]

    col96 = lax.broadcasted_iota(jnp.int32, (1, _C), 1)
    is_act_col = col96 < _ACT_V
    tau_col = jnp.where(is_act_col, tau_a, tau_t)
    s_tied_col = jnp.where(is_act_col, s_ta, s_tt)
    s_proto_col = jnp.where(is_act_col, s_pa, s_pt)

    @pl.when(b == 0)
    def _init_once():
        rowi = lax.broadcasted_iota(jnp.int32, (blk, blk), 0)
        coli = lax.broadcasted_iota(jnp.int32, (blk, blk), 1)
        strict = coli < rowi
        l_ref[...] = jnp.where(strict, 1.0, 0.0)
        lb_ref[...] = l_ref[...].astype(jnp.bfloat16)
        w_ref[_C:2 * _C, :] = e_ref[...]

    e = e_ref[...]
    en = e / jnp.maximum(
        jnp.sqrt(jnp.sum(e * e, axis=1, keepdims=True)), 1e-12)
    row_idx = lax.broadcasted_iota(jnp.int32, (_C, 1), 0)
    alpha_row = jnp.where(row_idx < _ACT_V, alpha_a, alpha_t)
    w_ref[0:_C, :] = alpha_row * en
    col128 = lax.broadcasted_iota(jnp.int32, (1, 128), 1)
    alpha_col = jnp.where(col128 < _ACT_V, alpha_a, alpha_t)
    sq_ref[...] = jnp.where(col128 < _C, alpha_col * alpha_col, 0.0)

    ones_l = l_ref[...]

    for j in range(t_len // blk):
        sl = slice(j * blk, (j + 1) * blk)
        h = h_ref[0, sl, :]
        hss = jnp.sum(h * h, axis=1, keepdims=True)
        inv = 1.0 / jnp.maximum(jnp.sqrt(hss), 1e-12)
        hn_ss = hss * inv * inv

        tok = tok_ref[0, sl, :]
        nxt = nxt_ref[0, sl, :]
        is_label = tok == _LABEL_ID
        sup_a = is_label & (nxt >= _SPECIAL) & (nxt < _SPECIAL + _ACT_V)
        sup_t = is_label & (nxt >= _SPECIAL + _ACT_V) & (nxt < _SPECIAL + _C)
        sup = sup_a | sup_t
        colc = lax.broadcasted_iota(jnp.int32, (blk, _C), 1)
        m = jnp.where(sup & (colc == nxt - _SPECIAL), 1.0, 0.0)
        mi = m * inv

        y = lax.dot_general(h, w_ref[...], (((1,), (1,)), ((), ())),
                            precision=_PREC)
        tied = y[:, _C:2 * _C]
        hb = h.astype(jnp.bfloat16)
        mib = mi.astype(jnp.bfloat16)
        gram = lax.dot_general(hb, hb, (((1,), (1,)), ((), ())),
                               preferred_element_type=jnp.float32,
                               precision=_PREC)
        gram_lb = (gram * ones_l).astype(jnp.bfloat16)
        num = inv * (y[:, :_C] +
                     lax.dot_general(gram_lb, mib, (((1,), (0,)), ((), ())),
                                     preferred_element_type=jnp.float32,
                                     precision=_PREC))

        inc = m * (2.0 * num + hn_ss)
        sup_af = jnp.where(sup_a, 1.0, 0.0)
        sup_tf = jnp.where(sup_t, 1.0, 0.0)
        x = jnp.concatenate(
            [inc, sup_af, sup_tf,
             jnp.zeros((blk, 128 - _C - 2), jnp.float32)],
            axis=1)
        cum = lax.dot_general(lb_ref[...], x.astype(jnp.bfloat16),
                              (((1,), (0,)), ((), ())),
                              preferred_element_type=jnp.float32,
                              precision=_PREC)
        base = sq_ref[...]
        sqnorm = base[:, :_C] + cum[:, :_C]
        cnts = base[:, _C:_C + 2] + cum[:, _C:_C + 2]

        denom = jnp.maximum(jnp.sqrt(jnp.maximum(sqnorm, 0.0)), 1e-12)
        gate_cnt = jnp.where(is_act_col, cnts[:, 0:1], cnts[:, 1:2])
        gate = is_label & (gate_cnt > 0.0)
        proto = jnp.where(gate, num / denom * tau_col, 0.0)

        out_ref[0, sl, :] = s_tied_col * tied + s_proto_col * proto

        w_ref[0:_C, :] = w_ref[0:_C, :] + lax.dot_general(
            mib, hb, (((0,), (0,)), ((), ())),
            preferred_element_type=jnp.float32, precision=_PREC)
        sq_ref[...] = base + jnp.sum(x, axis=0, keepdims=True)


def kernel(h, E, tokens, tied_scale_act, tied_scale_time, proto_scale_act,
           proto_scale_time, proto_prior_act, proto_prior_time,
           proto_temp_act, proto_temp_time):
    b, t, d = h.shape

    params = jnp.stack([
        jax.nn.softplus(tied_scale_act),
        jax.nn.softplus(tied_scale_time),
        jax.nn.softplus(proto_scale_act),
        jax.nn.softplus(proto_scale_time),
        jax.nn.softplus(proto_prior_act),
        jax.nn.softplus(proto_prior_time),
        jax.nn.softplus(proto_temp_act),
        jax.nn.softplus(proto_temp_time),
    ]).astype(jnp.float32)

    tokens = tokens.astype(jnp.int32)
    nxt = jnp.roll(tokens, -1, axis=1)
    tok3 = tokens.reshape(b, t, 1)
    nxt3 = nxt.reshape(b, t, 1)
    e_sub = E[_SPECIAL:_SPECIAL + _C].astype(jnp.float32)

    out = pl.pallas_call(
        _proto_kernel,
        grid=(b,),
        in_specs=[
            pl.BlockSpec((1, t, 1), lambda i: (i, 0, 0)),
            pl.BlockSpec((1, t, 1), lambda i: (i, 0, 0)),
            pl.BlockSpec((1, t, d), lambda i: (i, 0, 0)),
            pl.BlockSpec((_C, d), lambda i: (0, 0)),
            pl.BlockSpec(memory_space=pltpu.SMEM),
        ],
        out_specs=pl.BlockSpec((1, t, _C), lambda i: (i, 0, 0)),
        out_shape=jax.ShapeDtypeStruct((b, t, _C), jnp.float32),
        scratch_shapes=[
            pltpu.VMEM((2 * _C, d), jnp.float32),
            pltpu.VMEM((1, 128), jnp.float32),
            pltpu.VMEM((_BLK, _BLK), jnp.float32),
            pltpu.VMEM((_BLK, _BLK), jnp.bfloat16),
        ],
    )(tok3, nxt3, h.astype(jnp.float32), e_sub, params)
    return out
